# baseline (speedup 1.0000x reference)
import functools

import jax
import jax.numpy as jnp
from jax import lax
from jax.experimental import pallas as pl
from jax.experimental.pallas import tpu as pltpu
from jax.experimental.pallas import tpu_sc as plsc

NUM_LATENTS = 1000000
Z_DIM = 64
BATCH = 16384

NC, NS = 2, 16
NW = NC * NS
B_PER_W = BATCH // NW
BLK = 16
NBLK = B_PER_W // BLK
DEPTH = 4


def _gather_kernel(zr_hbm, idx_hbm, out_hbm, idx_v, sem):
    wid = lax.axis_index("s") * NC + lax.axis_index("c")
    base = wid * B_PER_W
    pltpu.sync_copy(idx_hbm.at[pl.ds(base, B_PER_W)], idx_v)

    def body(b, carry):
        v = idx_v[pl.ds(b * BLK, BLK)]
        for j in range(BLK):
            c = v[j]
            pltpu.async_copy(zr_hbm.at[c], out_hbm.at[base + b * BLK + j], sem)

        @pl.when(b >= DEPTH)
        def _drain_block():
            pltpu.make_async_copy(
                zr_hbm.at[pl.ds(0, BLK)],
                out_hbm.at[pl.ds(base + (b - DEPTH) * BLK, BLK)],
                sem,
            ).wait()

        return carry

    lax.fori_loop(0, NBLK, body, 0)
    pltpu.make_async_copy(
        zr_hbm.at[pl.ds(0, DEPTH * BLK)],
        out_hbm.at[pl.ds(base + (NBLK - DEPTH) * BLK, DEPTH * BLK)],
        sem,
    ).wait()


@jax.jit
def kernel(Z, indices):
    idx = indices.astype(jnp.int32)
    mesh = plsc.VectorSubcoreMesh(
        core_axis_name="c", subcore_axis_name="s",
        num_cores=NC, num_subcores=NS,
    )
    run = pl.kernel(
        _gather_kernel,
        out_type=jax.ShapeDtypeStruct((BATCH, Z_DIM), jnp.float32),
        mesh=mesh,
        scratch_types=[
            pltpu.VMEM((B_PER_W,), jnp.int32),
            pltpu.SemaphoreType.DMA,
        ],
    )
    return run(Z, idx)


# speedup vs baseline: 1.0844x; 1.0844x over previous
import functools

import jax
import jax.numpy as jnp
from jax import lax
from jax.experimental import pallas as pl
from jax.experimental.pallas import tpu as pltpu
from jax.experimental.pallas import tpu_sc as plsc

NUM_LATENTS = 1000000
Z_DIM = 64
PAD_DIM = 128
BATCH = 16384

NC, NS = 2, 16
NW = NC * NS
B_PER_W = BATCH // NW
CHUNK = 128
NCHUNK = B_PER_W // CHUNK


def _gather_kernel(zw_hbm, idx_hbm, out_hbm, idx_v, rows_v, sem):
    wid = lax.axis_index("s") * NC + lax.axis_index("c")
    base = wid * B_PER_W
    pltpu.sync_copy(idx_hbm.at[pl.ds(base, B_PER_W)], idx_v)
    for j in range(NCHUNK):
        pltpu.async_copy(
            zw_hbm.at[idx_v.at[pl.ds(j * CHUNK, CHUNK)]],
            rows_v.at[pl.ds(j * CHUNK, CHUNK), :],
            sem,
        )
    pltpu.make_async_copy(zw_hbm.at[pl.ds(0, B_PER_W)], rows_v, sem).wait()
    pltpu.sync_copy(rows_v, out_hbm.at[pl.ds(base, B_PER_W), :])


@jax.jit
def kernel(Z, indices):
    idx = indices.astype(jnp.int32)
    Zwide = lax.dynamic_update_slice(
        jnp.zeros((NUM_LATENTS, PAD_DIM), jnp.float32), Z, (0, 0)
    )
    mesh = plsc.VectorSubcoreMesh(
        core_axis_name="c", subcore_axis_name="s",
        num_cores=NC, num_subcores=NS,
    )
    run = pl.kernel(
        _gather_kernel,
        out_type=jax.ShapeDtypeStruct((BATCH, PAD_DIM), jnp.float32),
        mesh=mesh,
        scratch_types=[
            pltpu.VMEM((B_PER_W,), jnp.int32),
            pltpu.VMEM((B_PER_W, PAD_DIM), jnp.float32),
            pltpu.SemaphoreType.DMA,
        ],
    )
    return run(Zwide, idx)[:, :Z_DIM]
